# baseline (speedup 1.0000x reference)
import functools

import jax
import jax.numpy as jnp
from jax import lax
from jax.experimental import pallas as pl
from jax.experimental.pallas import tpu as pltpu
from jax.experimental.pallas import tpu_sc as plsc

_NUM_CP = 10
_FEAT = 128
_BATCH = 16384

_NC, _NS, _L = 2, 16, 16
_NW = _NC * _NS
_ROWS_PER_TILE = _BATCH // _NW
_CSIZES = (64, 192, 192, 64)
_COFFS = (0, 64, 256, 448)
_NCHUNK = len(_CSIZES)
_SCALE = float(_NUM_CP * (1.0 - 1e-06))

_mesh = plsc.VectorSubcoreMesh(
    core_axis_name="c", subcore_axis_name="s", num_cores=_NC, num_subcores=_NS
)


@functools.partial(
    pl.kernel,
    out_type=jax.ShapeDtypeStruct((_BATCH, _FEAT), jnp.float32),
    mesh=_mesh,
    scratch_types=[
        pltpu.VMEM((_NUM_CP * _FEAT,), jnp.float32),
        pltpu.VMEM(((_NUM_CP + 1) * _FEAT,), jnp.float32),
        *[pltpu.VMEM((n, _FEAT), jnp.float32) for n in _CSIZES],
        *[pltpu.SemaphoreType.DMA for _ in range(2 * len(_CSIZES))],
    ],
    compiler_params=pltpu.CompilerParams(needs_layout_passes=False),
)
def _curve_map_sc(x_hbm, cp_hbm, out_hbm, cp_v, curve_v, *bufs_and_sems):
    bufs = bufs_and_sems[:_NCHUNK]
    in_sems = bufs_and_sems[_NCHUNK:2 * _NCHUNK]
    out_sems = bufs_and_sems[2 * _NCHUNK:]
    wid = lax.axis_index("s") * _NC + lax.axis_index("c")
    iota = lax.iota(jnp.int32, _L)
    row0 = wid * _ROWS_PER_TILE

    h_in = [None] * _NCHUNK
    h_out = [None] * _NCHUNK

    def start_in(ch):
        h_in[ch] = pltpu.async_copy(
            x_hbm.at[pl.ds(row0 + _COFFS[ch], _CSIZES[ch])], bufs[ch],
            in_sems[ch])

    start_in(0)
    start_in(1)

    pltpu.sync_copy(cp_hbm, cp_v)

    @plsc.parallel_loop(0, _FEAT // _L)
    def col_body(c):
        col = c * _L
        curve_v[pl.ds(col, _L)] = jnp.zeros((_L,), jnp.float32)

        def cum_body(r, acc):
            acc = acc + jnp.exp(cp_v[pl.ds(r * _FEAT + col, _L)])
            curve_v[pl.ds((r + 1) * _FEAT + col, _L)] = acc
            return acc

        total = lax.fori_loop(0, _NUM_CP, cum_body, jnp.zeros((_L,), jnp.float32))
        inv = 1.0 / total

        def norm_body(r, _):
            off = r * _FEAT + col
            curve_v[pl.ds(off, _L)] = curve_v[pl.ds(off, _L)] * inv
            return 0

        lax.fori_loop(1, _NUM_CP + 1, norm_body, 0)

    curve_hi = curve_v.at[pl.ds(_FEAT, _NUM_CP * _FEAT)]
    for ch in range(_NCHUNK):
        buf = bufs[ch]
        h_in[ch].wait()
        if ch + 2 < _NCHUNK:
            start_in(ch + 2)

        @plsc.parallel_loop(0, _CSIZES[ch], unroll=2)
        def row_body(r):
            for c in range(_FEAT // _L):
                col = c * _L
                xv = buf[r, pl.ds(col, _L)] * _SCALE
                idx = xv.astype(jnp.int32)
                w = xv - idx.astype(jnp.float32)
                t = idx * _FEAT + (iota + col)
                lo = plsc.load_gather(curve_v, [t])
                hi = plsc.load_gather(curve_hi, [t])
                buf[r, pl.ds(col, _L)] = lo + w * (hi - lo)

        h_out[ch] = pltpu.async_copy(
            buf, out_hbm.at[pl.ds(row0 + _COFFS[ch], _CSIZES[ch])], out_sems[ch])

    for ch in range(_NCHUNK):
        h_out[ch].wait()


def kernel(inputs, control_points):
    return _curve_map_sc(inputs, control_points.reshape(-1))


# speedup vs baseline: 1.0113x; 1.0113x over previous
import functools

import jax
import jax.numpy as jnp
from jax import lax
from jax.experimental import pallas as pl
from jax.experimental.pallas import tpu as pltpu
from jax.experimental.pallas import tpu_sc as plsc

_NUM_CP = 10
_FEAT = 128
_BATCH = 16384

_NC, _NS, _L = 2, 16, 16
_NW = _NC * _NS
_ROWS_PER_TILE = _BATCH // _NW
_CSIZES = (64, 192, 192, 64)
_COFFS = (0, 64, 256, 448)
_NCHUNK = len(_CSIZES)
_SCALE = float(_NUM_CP * (1.0 - 1e-06))

_mesh = plsc.VectorSubcoreMesh(
    core_axis_name="c", subcore_axis_name="s", num_cores=_NC, num_subcores=_NS
)


@functools.partial(
    pl.kernel,
    out_type=jax.ShapeDtypeStruct((_BATCH, _FEAT), jnp.float32),
    mesh=_mesh,
    scratch_types=[
        pltpu.VMEM((_NUM_CP * _FEAT,), jnp.float32),
        pltpu.VMEM(((_NUM_CP + 1) * _FEAT,), jnp.float32),
        *[pltpu.VMEM((n, _FEAT), jnp.float32) for n in _CSIZES],
        *[pltpu.SemaphoreType.DMA for _ in range(2 * len(_CSIZES))],
    ],
    compiler_params=pltpu.CompilerParams(needs_layout_passes=False),
)
def _curve_map_sc(x_hbm, cp_hbm, out_hbm, cp_v, curve_v, *bufs_and_sems):
    bufs = bufs_and_sems[:_NCHUNK]
    in_sems = bufs_and_sems[_NCHUNK:2 * _NCHUNK]
    out_sems = bufs_and_sems[2 * _NCHUNK:]
    wid = lax.axis_index("s") * _NC + lax.axis_index("c")
    iota = lax.iota(jnp.int32, _L)
    row0 = wid * _ROWS_PER_TILE

    h_in = [None] * _NCHUNK
    h_out = [None] * _NCHUNK

    def start_in(ch):
        h_in[ch] = pltpu.async_copy(
            x_hbm.at[pl.ds(row0 + _COFFS[ch], _CSIZES[ch])], bufs[ch],
            in_sems[ch])

    start_in(0)
    start_in(1)

    pltpu.sync_copy(cp_hbm, cp_v)

    @plsc.parallel_loop(0, _FEAT // _L)
    def col_body(c):
        col = c * _L
        curve_v[pl.ds(col, _L)] = jnp.zeros((_L,), jnp.float32)

        def cum_body(r, acc):
            acc = acc + jnp.exp(cp_v[pl.ds(r * _FEAT + col, _L)])
            curve_v[pl.ds((r + 1) * _FEAT + col, _L)] = acc
            return acc

        total = lax.fori_loop(0, _NUM_CP, cum_body, jnp.zeros((_L,), jnp.float32))
        inv = 1.0 / total

        def norm_body(r, _):
            off = r * _FEAT + col
            curve_v[pl.ds(off, _L)] = curve_v[pl.ds(off, _L)] * inv
            return 0

        lax.fori_loop(1, _NUM_CP + 1, norm_body, 0)

    curve_hi = curve_v.at[pl.ds(_FEAT, _NUM_CP * _FEAT)]
    for ch in range(_NCHUNK):
        buf = bufs[ch]
        h_in[ch].wait()
        if ch + 2 < _NCHUNK:
            start_in(ch + 2)

        @plsc.parallel_loop(0, _CSIZES[ch], unroll=4)
        def row_body(r):
            for c in range(_FEAT // _L):
                col = c * _L
                xv = buf[r, pl.ds(col, _L)] * _SCALE
                idx = xv.astype(jnp.int32)
                w = xv - idx.astype(jnp.float32)
                t = idx * _FEAT + (iota + col)
                lo = plsc.load_gather(curve_v, [t])
                hi = plsc.load_gather(curve_hi, [t])
                buf[r, pl.ds(col, _L)] = lo + w * (hi - lo)

        h_out[ch] = pltpu.async_copy(
            buf, out_hbm.at[pl.ds(row0 + _COFFS[ch], _CSIZES[ch])], out_sems[ch])

    for ch in range(_NCHUNK):
        h_out[ch].wait()


def kernel(inputs, control_points):
    return _curve_map_sc(inputs, control_points.reshape(-1))
